# Initial kernel scaffold; baseline (speedup 1.0000x reference)
import functools

import jax
import jax.numpy as jnp
from jax import lax
from jax.experimental import pallas as pl
from jax.experimental.pallas import tpu as pltpu
from jax.experimental.pallas import tpu_sc as plsc

DIM = 64
NUM_CORES = 2
NUM_SUBCORES = 16
NW = NUM_CORES * NUM_SUBCORES
LANES = 16
GROUP = 128


def _body(idx_hbm, ktab_hbm, pret_hbm, out_hbm, idx_v, sidx_v, ktab_v, rows_v):
    n = idx_hbm.shape[0]
    bpw = n // NW
    wid = lax.axis_index("s") * NUM_CORES + lax.axis_index("c")
    base = wid * bpw

    pltpu.sync_copy(idx_hbm.at[pl.ds(base, bpw)], idx_v)
    pltpu.sync_copy(ktab_hbm, ktab_v)

    def _mk_safe(i, carry):
        v = idx_v[pl.ds(i * LANES, LANES)]
        sidx_v[pl.ds(i * LANES, LANES)] = jnp.maximum(v - 2, 0)
        return carry

    lax.fori_loop(0, bpw // LANES, _mk_safe, 0)

    lane = lax.iota(jnp.int32, LANES)

    def _group(g, carry):
        gbase = g * GROUP
        pltpu.sync_copy(pret_hbm.at[sidx_v.at[pl.ds(gbase, GROUP)]], rows_v)

        def _fix(j, c):
            v = idx_v[pl.ds(gbase + j * LANES, LANES)]

            @pl.when(jnp.min(v) < 2)
            def _():
                m = v < 2
                vsafe = jnp.minimum(v, 1)
                r16 = lane + j * LANES
                for dcol in range(DIM):
                    col = jnp.full((LANES,), dcol, jnp.int32)
                    x = plsc.load_gather(ktab_v, [vsafe, col])
                    plsc.store_scatter(rows_v, [r16, col], x, m)

            return c

        lax.fori_loop(0, GROUP // LANES, _fix, 0)

        pltpu.sync_copy(rows_v, out_hbm.at[pl.ds(base + gbase, GROUP)])
        return carry

    lax.fori_loop(0, bpw // GROUP, _group, 0)


def _gather(idx, ktab, pret):
    n = idx.shape[0]
    mesh = plsc.VectorSubcoreMesh(core_axis_name="c", subcore_axis_name="s")
    bpw = n // NW
    return pl.kernel(
        _body,
        out_type=jax.ShapeDtypeStruct((n, DIM), jnp.float32),
        mesh=mesh,
        scratch_types=[
            pltpu.VMEM((bpw,), jnp.int32),
            pltpu.VMEM((bpw,), jnp.int32),
            pltpu.VMEM((2, DIM), jnp.float32),
            pltpu.VMEM((GROUP, DIM), jnp.float32),
        ],
    )(idx, ktab, pret)


def kernel(inputs, kernel, pretrained):
    idx = inputs.reshape(-1).astype(jnp.int32)
    out = _gather(idx, kernel, pretrained)
    return out.reshape(inputs.shape + (DIM,))


# speedup vs baseline: 4.0309x; 4.0309x over previous
import functools

import jax
import jax.numpy as jnp
from jax import lax
from jax.experimental import pallas as pl
from jax.experimental.pallas import tpu as pltpu
from jax.experimental.pallas import tpu_sc as plsc

DIM = 64
NUM_CORES = 2
NUM_SUBCORES = 16
NW = NUM_CORES * NUM_SUBCORES
LANES = 16
GROUP = 128


def _body(idx_hbm, ktab_hbm, pret_hbm, out_hbm, idx_v, sidx_v, ktab_v, rows_v):
    n = idx_hbm.shape[0]
    bpw = n // NW
    wid = lax.axis_index("s") * NUM_CORES + lax.axis_index("c")
    base = wid * bpw

    pltpu.sync_copy(idx_hbm.at[pl.ds(base, bpw)], idx_v)
    pltpu.sync_copy(ktab_hbm, ktab_v)

    def _mk_safe(i, carry):
        v = idx_v[pl.ds(i * LANES, LANES)]
        sidx_v[pl.ds(i * LANES, LANES)] = jnp.maximum(v - 2, 0)
        return carry

    lax.fori_loop(0, bpw // LANES, _mk_safe, 0)

    lane = lax.iota(jnp.int32, LANES)

    def _group(g, carry):
        gbase = g * GROUP
        pltpu.sync_copy(pret_hbm.at[sidx_v.at[pl.ds(gbase, GROUP)]], rows_v)

        def _fix(j, c):
            v = idx_v[pl.ds(gbase + j * LANES, LANES)]

            @pl.when(jnp.min(v) < 2)
            def _():
                m = v < 2
                vsafe = jnp.minimum(v, 1)
                r16 = lane + j * LANES
                for dcol in range(DIM):
                    col = jnp.full((LANES,), dcol, jnp.int32)
                    x = plsc.load_gather(ktab_v, [vsafe, col])
                    plsc.store_scatter(rows_v, [r16, col], x, mask=m)

            return c

        lax.fori_loop(0, GROUP // LANES, _fix, 0)

        pltpu.sync_copy(rows_v, out_hbm.at[pl.ds(base + gbase, GROUP)])
        return carry

    lax.fori_loop(0, bpw // GROUP, _group, 0)


def _gather(idx, ktab, pret):
    n = idx.shape[0]
    mesh = plsc.VectorSubcoreMesh(core_axis_name="c", subcore_axis_name="s")
    bpw = n // NW
    return pl.kernel(
        _body,
        out_type=jax.ShapeDtypeStruct((n, DIM), jnp.float32),
        mesh=mesh,
        compiler_params=pltpu.CompilerParams(
            needs_layout_passes=False, use_tc_tiling_on_sc=False
        ),
        scratch_types=[
            pltpu.VMEM((bpw,), jnp.int32),
            pltpu.VMEM((bpw,), jnp.int32),
            pltpu.VMEM((2, DIM), jnp.float32),
            pltpu.VMEM((GROUP, DIM), jnp.float32),
        ],
    )(idx, ktab, pret)


def kernel(inputs, kernel, pretrained):
    idx = inputs.reshape(-1).astype(jnp.int32)
    out = _gather(idx, kernel, pretrained)
    return out.reshape(inputs.shape + (DIM,))


# speedup vs baseline: 4.6833x; 1.1619x over previous
import functools

import jax
import jax.numpy as jnp
from jax import lax
from jax.experimental import pallas as pl
from jax.experimental.pallas import tpu as pltpu
from jax.experimental.pallas import tpu_sc as plsc

DIM = 64
NUM_CORES = 2
NUM_SUBCORES = 16
NW = NUM_CORES * NUM_SUBCORES
LANES = 16
GROUP = 128


CHUNK = 640
SUB = CHUNK // GROUP


def _body(idx_hbm, ktab_hbm, pret_hbm, out_hbm, idx_v, sidx_v, ktab_v,
          rows0, rows1, gsem0, gsem1, wsem0, wsem1):
    n = idx_hbm.shape[0]
    bpw = n // NW
    nchunk = bpw // CHUNK
    wid = lax.axis_index("s") * NUM_CORES + lax.axis_index("c")
    base = wid * bpw
    rows = (rows0, rows1)
    gsem = (gsem0, gsem1)
    wsem = (wsem0, wsem1)

    pltpu.sync_copy(idx_hbm.at[pl.ds(base, bpw)], idx_v)
    pltpu.sync_copy(ktab_hbm, ktab_v)

    def _mk_safe(i, carry):
        v = idx_v[pl.ds(i * LANES, LANES)]
        sidx_v[pl.ds(i * LANES, LANES)] = jnp.maximum(v - 2, 0)
        return carry

    lax.fori_loop(0, bpw // LANES, _mk_safe, 0)

    lane = lax.iota(jnp.int32, LANES)

    def _fire_gathers(c, b):
        return [
            pltpu.async_copy(
                pret_hbm.at[sidx_v.at[pl.ds(c * CHUNK + s * GROUP, GROUP)]],
                rows[b].at[pl.ds(s * GROUP, GROUP)],
                gsem[b],
            )
            for s in range(SUB)
        ]

    def _fixup(c, b):
        def _fix(j, cc):
            v = idx_v[pl.ds(c * CHUNK + j * LANES, LANES)]

            @pl.when(jnp.min(v) < 2)
            def _():
                m = v < 2
                vsafe = jnp.minimum(v, 1)
                r16 = lane + j * LANES
                for dcol in range(DIM):
                    col = jnp.full((LANES,), dcol, jnp.int32)
                    x = plsc.load_gather(ktab_v, [vsafe, col])
                    plsc.store_scatter(rows[b], [r16, col], x, mask=m)

            return cc

        lax.fori_loop(0, CHUNK // LANES, _fix, 0)

    g_pending = _fire_gathers(0, 0)
    w_pending = [None] * nchunk
    for i in range(nchunk):
        b = i % 2
        if i + 1 < nchunk:
            if i >= 1:
                w_pending[i - 1].wait()
            g_next = _fire_gathers(i + 1, 1 - b)
        for h in g_pending:
            h.wait()
        _fixup(i, b)
        w_pending[i] = pltpu.async_copy(
            rows[b], out_hbm.at[pl.ds(base + i * CHUNK, CHUNK)], wsem[b]
        )
        if i + 1 < nchunk:
            g_pending = g_next
    w_pending[nchunk - 2].wait()
    w_pending[nchunk - 1].wait()


def _gather(idx, ktab, pret):
    n = idx.shape[0]
    mesh = plsc.VectorSubcoreMesh(core_axis_name="c", subcore_axis_name="s")
    bpw = n // NW
    return pl.kernel(
        _body,
        out_type=jax.ShapeDtypeStruct((n, DIM), jnp.float32),
        mesh=mesh,
        compiler_params=pltpu.CompilerParams(
            needs_layout_passes=False, use_tc_tiling_on_sc=False
        ),
        scratch_types=[
            pltpu.VMEM((bpw,), jnp.int32),
            pltpu.VMEM((bpw,), jnp.int32),
            pltpu.VMEM((2, DIM), jnp.float32),
            pltpu.VMEM((CHUNK, DIM), jnp.float32),
            pltpu.VMEM((CHUNK, DIM), jnp.float32),
            pltpu.SemaphoreType.DMA,
            pltpu.SemaphoreType.DMA,
            pltpu.SemaphoreType.DMA,
            pltpu.SemaphoreType.DMA,
        ],
    )(idx, ktab, pret)


def kernel(inputs, kernel, pretrained):
    idx = inputs.reshape(-1).astype(jnp.int32)
    out = _gather(idx, kernel, pretrained)
    return out.reshape(inputs.shape + (DIM,))
